# Initial kernel scaffold; baseline (speedup 1.0000x reference)
import jax
import jax.numpy as jnp
from jax.experimental import pallas as pl


def kernel(bands, enc_w1, enc_b1, enc_w2, enc_b2, codebooks, dec_w1, dec_b1, dec_w2, dec_b2):
    raise NotImplementedError("write your pallas kernel here")


# speedup vs baseline: 4.1534x; 4.1534x over previous
import functools

import jax
import jax.numpy as jnp
from jax.experimental import pallas as pl


def _fused_body(x_ref, w1_ref, b1_ref, w2_ref, b2_ref, cb_ref,
                dw1_ref, db1_ref, dw2_ref, db2_ref,
                bh_ref, ze_ref, zq_ref, idx_ref, loss_ref,
                *, G, K, GD):
    x = x_ref[...]

    h = jnp.maximum(jnp.dot(x, w1_ref[...]) + b1_ref[...], 0.0)
    z_e = jnp.dot(h, w2_ref[...]) + b2_ref[...]
    ze_ref[...] = z_e

    zq_parts = []
    idx_parts = []
    block_sq = jnp.zeros((), dtype=jnp.float32)
    for g in range(G):
        zg = z_e[:, g * GD:(g + 1) * GD]
        cbg = cb_ref[g]
        z_norm = jnp.sum(zg * zg, axis=1, keepdims=True)
        cb_norm = jnp.sum(cbg * cbg, axis=1)[None, :]
        scores = jax.lax.dot_general(zg, cbg, (((1,), (1,)), ((), ())))
        dist = z_norm - 2.0 * scores + cb_norm
        m = jnp.min(dist, axis=1, keepdims=True)
        iota = jax.lax.broadcasted_iota(jnp.int32, dist.shape, 1)
        idxg = jnp.min(jnp.where(dist == m, iota, K), axis=1, keepdims=True)
        onehot = (iota == idxg).astype(jnp.float32)
        zqg = jnp.dot(onehot, cbg)
        d = zqg - zg
        block_sq = block_sq + jnp.sum(d * d)
        zq_parts.append(zqg)
        idx_parts.append(idxg)

    z_q = jnp.concatenate(zq_parts, axis=1)
    zq_ref[...] = z_q
    idx_ref[...] = jnp.concatenate(idx_parts, axis=1)

    @pl.when(pl.program_id(0) == 0)
    def _init():
        loss_ref[...] = jnp.zeros((1, 1), jnp.float32)

    loss_ref[...] += block_sq[None, None]

    hd = jnp.maximum(jnp.dot(z_q, dw1_ref[...]) + db1_ref[...], 0.0)
    bh_ref[...] = jnp.dot(hd, dw2_ref[...]) + db2_ref[...]


def kernel(bands, enc_w1, enc_b1, enc_w2, enc_b2, codebooks,
           dec_w1, dec_b1, dec_w2, dec_b2):
    B, T, D = bands.shape
    N = B * T
    G, K, GD = codebooks.shape
    H = enc_w1.shape[1]
    L = enc_w2.shape[1]
    beta = 0.25

    R = 512
    assert N % R == 0
    grid = (N // R,)

    x = bands.reshape(N, D)
    b1 = enc_b1.reshape(1, H)
    b2 = enc_b2.reshape(1, L)
    db1 = dec_b1.reshape(1, H)
    db2 = dec_b2.reshape(1, D)

    row_spec = lambda c: pl.BlockSpec((R, c), lambda i: (i, 0))
    full2 = lambda a, b: pl.BlockSpec((a, b), lambda i: (0, 0))

    out_shapes = (
        jax.ShapeDtypeStruct((N, D), jnp.float32),
        jax.ShapeDtypeStruct((N, L), jnp.float32),
        jax.ShapeDtypeStruct((N, L), jnp.float32),
        jax.ShapeDtypeStruct((N, G), jnp.int32),
        jax.ShapeDtypeStruct((1, 1), jnp.float32),
    )

    bands_hat, z_e, z_q, idx, loss_sum = pl.pallas_call(
        functools.partial(_fused_body, G=G, K=K, GD=GD),
        grid=grid,
        in_specs=[
            row_spec(D),
            full2(D, H), full2(1, H), full2(H, L), full2(1, L),
            pl.BlockSpec((G, K, GD), lambda i: (0, 0, 0)),
            full2(L, H), full2(1, H), full2(H, D), full2(1, D),
        ],
        out_specs=(
            row_spec(D), row_spec(L), row_spec(L),
            pl.BlockSpec((R, G), lambda i: (i, 0)),
            pl.BlockSpec((1, 1), lambda i: (0, 0)),
        ),
        out_shape=out_shapes,
    )(x, enc_w1, b1, enc_w2, b2, codebooks, dec_w1, db1, dec_w2, db2)

    vq_loss = (2.0 * beta / (N * GD)) * loss_sum[0, 0]
    return (bands_hat.reshape(B, T, D), z_e.reshape(B, T, L),
            z_q.reshape(B, T, L), idx.reshape(B, T, G), vq_loss)


# speedup vs baseline: 4.3982x; 1.0589x over previous
import functools

import jax
import jax.numpy as jnp
from jax.experimental import pallas as pl


def _fused_body(x_ref, w1_ref, b1_ref, w2_ref, b2_ref, cb_ref,
                dw1_ref, db1_ref, dw2_ref, db2_ref,
                bh_ref, ze_ref, zq_ref, idx_ref, loss_ref,
                *, G, K, GD):
    x = x_ref[...]

    h = jnp.maximum(jnp.dot(x, w1_ref[...]) + b1_ref[...], 0.0)
    z_e = jnp.dot(h, w2_ref[...]) + b2_ref[...]
    ze_ref[...] = z_e

    R = x.shape[0]
    iota_f = jax.lax.broadcasted_iota(jnp.int32, (R, K), 1).astype(jnp.float32)
    big = float(K)
    cbn_all = jnp.sum(cb_ref[...] * cb_ref[...], axis=2)

    zq_parts = []
    idx_parts = []
    for g in range(G):
        zg = z_e[:, g * GD:(g + 1) * GD]
        cbg = cb_ref[g]
        z_norm = jnp.sum(zg * zg, axis=1, keepdims=True)
        scores = jax.lax.dot_general(zg, cbg, (((1,), (1,)), ((), ())))
        dist = z_norm - 2.0 * scores + cbn_all[g][None, :]
        m = jnp.min(dist, axis=1, keepdims=True)
        eq = dist == m
        key = jnp.where(eq, iota_f, big)
        idxg = jnp.min(key, axis=1, keepdims=True)
        maskf = jnp.where(eq, 1.0, 0.0)
        zqg = jnp.dot(maskf, cbg)
        zq_parts.append(zqg)
        idx_parts.append(idxg)

    z_q = jnp.concatenate(zq_parts, axis=1)
    zq_ref[...] = z_q
    idx_ref[...] = jnp.concatenate(idx_parts, axis=1).astype(jnp.int32)
    dz = z_q - z_e
    block_sq = jnp.sum(dz * dz)

    @pl.when(pl.program_id(0) == 0)
    def _init():
        loss_ref[...] = jnp.zeros((1, 1), jnp.float32)

    loss_ref[...] += block_sq[None, None]

    hd = jnp.maximum(jnp.dot(z_q, dw1_ref[...]) + db1_ref[...], 0.0)
    bh_ref[...] = jnp.dot(hd, dw2_ref[...]) + db2_ref[...]


def kernel(bands, enc_w1, enc_b1, enc_w2, enc_b2, codebooks,
           dec_w1, dec_b1, dec_w2, dec_b2):
    B, T, D = bands.shape
    N = B * T
    G, K, GD = codebooks.shape
    H = enc_w1.shape[1]
    L = enc_w2.shape[1]
    beta = 0.25

    R = 512
    assert N % R == 0
    grid = (N // R,)

    x = bands.reshape(N, D)
    b1 = enc_b1.reshape(1, H)
    b2 = enc_b2.reshape(1, L)
    db1 = dec_b1.reshape(1, H)
    db2 = dec_b2.reshape(1, D)

    row_spec = lambda c: pl.BlockSpec((R, c), lambda i: (i, 0))
    full2 = lambda a, b: pl.BlockSpec((a, b), lambda i: (0, 0))

    out_shapes = (
        jax.ShapeDtypeStruct((N, D), jnp.float32),
        jax.ShapeDtypeStruct((N, L), jnp.float32),
        jax.ShapeDtypeStruct((N, L), jnp.float32),
        jax.ShapeDtypeStruct((N, G), jnp.int32),
        jax.ShapeDtypeStruct((1, 1), jnp.float32),
    )

    bands_hat, z_e, z_q, idx, loss_sum = pl.pallas_call(
        functools.partial(_fused_body, G=G, K=K, GD=GD),
        grid=grid,
        in_specs=[
            row_spec(D),
            full2(D, H), full2(1, H), full2(H, L), full2(1, L),
            pl.BlockSpec((G, K, GD), lambda i: (0, 0, 0)),
            full2(L, H), full2(1, H), full2(H, D), full2(1, D),
        ],
        out_specs=(
            row_spec(D), row_spec(L), row_spec(L),
            pl.BlockSpec((R, G), lambda i: (i, 0)),
            pl.BlockSpec((1, 1), lambda i: (0, 0)),
        ),
        out_shape=out_shapes,
    )(x, enc_w1, b1, enc_w2, b2, codebooks, dec_w1, db1, dec_w2, db2)

    vq_loss = (2.0 * beta / (N * GD)) * loss_sum[0, 0]
    return (bands_hat.reshape(B, T, D), z_e.reshape(B, T, L),
            z_q.reshape(B, T, L), idx.reshape(B, T, G), vq_loss)


# speedup vs baseline: 7.2368x; 1.6454x over previous
import functools

import jax
import jax.numpy as jnp
from jax.experimental import pallas as pl
from jax.experimental.pallas import tpu as pltpu


def _fused_body(x_ref, w1_ref, b1_ref, w2_ref, b2_ref, cb_ref,
                dw1_ref, db1_ref, dw2_ref, db2_ref,
                bh_ref, ze_ref, zq_ref, idx_ref, loss_ref,
                cb2_ref, cbn_ref, aug_ref,
                *, G, K, GD):
    @pl.when(pl.program_id(0) == 0)
    def _prep():
        cb = cb_ref[...]
        cb2_ref[...] = cb * 2.0
        cbn_ref[...] = jnp.sum(cb * cb, axis=2)
        aug_ref[:, :, :GD] = cb
        aug_ref[:, :, GD:] = jax.lax.broadcasted_iota(
            jnp.int32, (G, K, 1), 1).astype(jnp.float32)

    x = x_ref[...]

    h = jnp.maximum(jnp.dot(x, w1_ref[...]) + b1_ref[...], 0.0)
    z_e = jnp.dot(h, w2_ref[...]) + b2_ref[...]
    ze_ref[...] = z_e

    zq_parts = []
    idx_parts = []
    for g in range(G):
        zg = z_e[:, g * GD:(g + 1) * GD]
        s2 = jax.lax.dot_general(zg, cb2_ref[g], (((1,), (1,)), ((), ())))
        a = s2 - cbn_ref[g][None, :]
        m = jnp.max(a, axis=1, keepdims=True)
        maskf = jnp.where(a == m, 1.0, 0.0)
        r = jnp.dot(maskf, aug_ref[g])
        zq_parts.append(r[:, :GD])
        idx_parts.append(r[:, GD:])

    z_q = jnp.concatenate(zq_parts, axis=1)
    zq_ref[...] = z_q
    idx_ref[...] = jnp.concatenate(idx_parts, axis=1).astype(jnp.int32)
    dz = z_q - z_e
    block_sq = jnp.sum(dz * dz)

    @pl.when(pl.program_id(0) == 0)
    def _init():
        loss_ref[...] = jnp.zeros((1, 1), jnp.float32)

    loss_ref[...] += block_sq[None, None]

    hd = jnp.maximum(jnp.dot(z_q, dw1_ref[...]) + db1_ref[...], 0.0)
    bh_ref[...] = jnp.dot(hd, dw2_ref[...]) + db2_ref[...]


def kernel(bands, enc_w1, enc_b1, enc_w2, enc_b2, codebooks,
           dec_w1, dec_b1, dec_w2, dec_b2):
    B, T, D = bands.shape
    N = B * T
    G, K, GD = codebooks.shape
    H = enc_w1.shape[1]
    L = enc_w2.shape[1]
    beta = 0.25

    R = 512
    assert N % R == 0
    grid = (N // R,)

    x = bands.reshape(N, D)
    b1 = enc_b1.reshape(1, H)
    b2 = enc_b2.reshape(1, L)
    db1 = dec_b1.reshape(1, H)
    db2 = dec_b2.reshape(1, D)

    row_spec = lambda c: pl.BlockSpec((R, c), lambda i: (i, 0))
    full2 = lambda a, b: pl.BlockSpec((a, b), lambda i: (0, 0))

    out_shapes = (
        jax.ShapeDtypeStruct((N, D), jnp.float32),
        jax.ShapeDtypeStruct((N, L), jnp.float32),
        jax.ShapeDtypeStruct((N, L), jnp.float32),
        jax.ShapeDtypeStruct((N, G), jnp.int32),
        jax.ShapeDtypeStruct((1, 1), jnp.float32),
    )

    bands_hat, z_e, z_q, idx, loss_sum = pl.pallas_call(
        functools.partial(_fused_body, G=G, K=K, GD=GD),
        grid=grid,
        in_specs=[
            row_spec(D),
            full2(D, H), full2(1, H), full2(H, L), full2(1, L),
            pl.BlockSpec((G, K, GD), lambda i: (0, 0, 0)),
            full2(L, H), full2(1, H), full2(H, D), full2(1, D),
        ],
        out_specs=(
            row_spec(D), row_spec(L), row_spec(L),
            pl.BlockSpec((R, G), lambda i: (i, 0)),
            pl.BlockSpec((1, 1), lambda i: (0, 0)),
        ),
        out_shape=out_shapes,
        scratch_shapes=[
            pltpu.VMEM((G, K, GD), jnp.float32),
            pltpu.VMEM((G, K), jnp.float32),
            pltpu.VMEM((G, K, GD + 1), jnp.float32),
        ],
    )(x, enc_w1, b1, enc_w2, b2, codebooks, dec_w1, db1, dec_w2, db2)

    vq_loss = (2.0 * beta / (N * GD)) * loss_sum[0, 0]
    return (bands_hat.reshape(B, T, D), z_e.reshape(B, T, L),
            z_q.reshape(B, T, L), idx.reshape(B, T, G), vq_loss)


# speedup vs baseline: 7.9846x; 1.1033x over previous
import functools

import jax
import jax.numpy as jnp
from jax.experimental import pallas as pl
from jax.experimental.pallas import tpu as pltpu


def _fused_body(x_ref, w1_ref, b1_ref, w2_ref, b2_ref, cb_ref,
                dw1_ref, db1_ref, dw2_ref, db2_ref,
                bh_ref, ze_ref, zq_ref, idx_ref, loss_ref,
                cb2_ref, cbn_ref, aug_ref,
                *, G, K, GD):
    @pl.when(pl.program_id(0) == 0)
    def _prep():
        cb = cb_ref[...]
        cb2_ref[...] = cb * 2.0
        cbn_ref[...] = jnp.sum(cb * cb, axis=2)
        aug_ref[:, :, :GD] = cb
        aug_ref[:, :, GD:] = jax.lax.broadcasted_iota(
            jnp.int32, (G, K, 1), 1).astype(jnp.float32)

    x = x_ref[...]

    h = jnp.maximum(jnp.dot(x, w1_ref[...]) + b1_ref[...], 0.0)
    z_e = jnp.dot(h, w2_ref[...]) + b2_ref[...]
    ze_ref[...] = z_e

    zq_parts = []
    idx_parts = []
    for g in range(G):
        zg = z_e[:, g * GD:(g + 1) * GD]
        s2 = jax.lax.dot_general(zg, cb2_ref[g], (((1,), (1,)), ((), ())))
        a = s2 - cbn_ref[g][None, :]
        m = jnp.max(a, axis=1, keepdims=True)
        maskf = jnp.where(a == m, 1.0, 0.0)
        r = jnp.dot(maskf, aug_ref[g])
        zq_parts.append(r[:, :GD])
        idx_parts.append(r[:, GD:])

    z_q = jnp.concatenate(zq_parts, axis=1)
    zq_ref[...] = z_q
    idx_ref[...] = jnp.concatenate(idx_parts, axis=1).astype(jnp.int32)
    dz = z_q - z_e
    block_sq = jnp.sum(dz * dz)

    @pl.when(pl.program_id(0) == 0)
    def _init():
        loss_ref[...] = jnp.zeros((1, 1), jnp.float32)

    loss_ref[...] += block_sq[None, None]

    hd = jnp.maximum(jnp.dot(z_q, dw1_ref[...]) + db1_ref[...], 0.0)
    bh_ref[...] = jnp.dot(hd, dw2_ref[...]) + db2_ref[...]


def kernel(bands, enc_w1, enc_b1, enc_w2, enc_b2, codebooks,
           dec_w1, dec_b1, dec_w2, dec_b2):
    B, T, D = bands.shape
    N = B * T
    G, K, GD = codebooks.shape
    H = enc_w1.shape[1]
    L = enc_w2.shape[1]
    beta = 0.25

    R = 1024
    assert N % R == 0
    grid = (N // R,)

    x = bands.reshape(N, D)
    b1 = enc_b1.reshape(1, H)
    b2 = enc_b2.reshape(1, L)
    db1 = dec_b1.reshape(1, H)
    db2 = dec_b2.reshape(1, D)

    row_spec = lambda c: pl.BlockSpec((R, c), lambda i: (i, 0))
    full2 = lambda a, b: pl.BlockSpec((a, b), lambda i: (0, 0))

    out_shapes = (
        jax.ShapeDtypeStruct((N, D), jnp.float32),
        jax.ShapeDtypeStruct((N, L), jnp.float32),
        jax.ShapeDtypeStruct((N, L), jnp.float32),
        jax.ShapeDtypeStruct((N, G), jnp.int32),
        jax.ShapeDtypeStruct((1, 1), jnp.float32),
    )

    bands_hat, z_e, z_q, idx, loss_sum = pl.pallas_call(
        functools.partial(_fused_body, G=G, K=K, GD=GD),
        grid=grid,
        in_specs=[
            row_spec(D),
            full2(D, H), full2(1, H), full2(H, L), full2(1, L),
            pl.BlockSpec((G, K, GD), lambda i: (0, 0, 0)),
            full2(L, H), full2(1, H), full2(H, D), full2(1, D),
        ],
        out_specs=(
            row_spec(D), row_spec(L), row_spec(L),
            pl.BlockSpec((R, G), lambda i: (i, 0)),
            pl.BlockSpec((1, 1), lambda i: (0, 0)),
        ),
        out_shape=out_shapes,
        scratch_shapes=[
            pltpu.VMEM((G, K, GD), jnp.float32),
            pltpu.VMEM((G, K), jnp.float32),
            pltpu.VMEM((G, K, GD + 1), jnp.float32),
        ],
    )(x, enc_w1, b1, enc_w2, b2, codebooks, dec_w1, db1, dec_w2, db2)

    vq_loss = (2.0 * beta / (N * GD)) * loss_sum[0, 0]
    return (bands_hat.reshape(B, T, D), z_e.reshape(B, T, L),
            z_q.reshape(B, T, L), idx.reshape(B, T, G), vq_loss)


# speedup vs baseline: 8.6240x; 1.0801x over previous
import functools

import jax
import jax.numpy as jnp
from jax.experimental import pallas as pl
from jax.experimental.pallas import tpu as pltpu


def _fused_body(x_ref, w1_ref, b1_ref, w2_ref, b2_ref, cb_ref,
                dw1_ref, db1_ref, dw2_ref, db2_ref,
                bh_ref, ze_ref, zq_ref, idx_ref, loss_ref,
                cb2_ref, cbn_ref, aug_ref,
                *, G, K, GD):
    @pl.when(pl.program_id(0) == 0)
    def _prep():
        cb = cb_ref[...]
        cb2_ref[...] = cb * 2.0
        cbn_ref[...] = jnp.sum(cb * cb, axis=2)
        aug_ref[:, :, :GD] = cb
        aug_ref[:, :, GD:] = jax.lax.broadcasted_iota(
            jnp.int32, (G, K, 1), 1).astype(jnp.float32)

    x = x_ref[...]

    h = jnp.maximum(jnp.dot(x, w1_ref[...]) + b1_ref[...], 0.0)
    z_e = jnp.dot(h, w2_ref[...]) + b2_ref[...]
    ze_ref[...] = z_e

    zq_parts = []
    idx_parts = []
    for g in range(G):
        zg = z_e[:, g * GD:(g + 1) * GD]
        s2 = jax.lax.dot_general(zg, cb2_ref[g], (((1,), (1,)), ((), ())))
        a = s2 - cbn_ref[g][None, :]
        m = jnp.max(a, axis=1, keepdims=True)
        maskf = jnp.where(a == m, 1.0, 0.0)
        r = jnp.dot(maskf, aug_ref[g])
        zq_parts.append(r[:, :GD])
        idx_parts.append(r[:, GD:])

    z_q = jnp.concatenate(zq_parts, axis=1)
    zq_ref[...] = z_q
    idx_ref[...] = jnp.concatenate(idx_parts, axis=1).astype(jnp.int32)
    dz = z_q - z_e
    block_sq = jnp.sum(dz * dz)

    @pl.when(pl.program_id(0) == 0)
    def _init():
        loss_ref[...] = jnp.zeros((1, 1), jnp.float32)

    loss_ref[...] += block_sq[None, None]

    hd = jnp.maximum(jnp.dot(z_q, dw1_ref[...]) + db1_ref[...], 0.0)
    bh_ref[...] = jnp.dot(hd, dw2_ref[...]) + db2_ref[...]


def kernel(bands, enc_w1, enc_b1, enc_w2, enc_b2, codebooks,
           dec_w1, dec_b1, dec_w2, dec_b2):
    B, T, D = bands.shape
    N = B * T
    G, K, GD = codebooks.shape
    H = enc_w1.shape[1]
    L = enc_w2.shape[1]
    beta = 0.25

    R = 2048
    assert N % R == 0
    grid = (N // R,)

    x = bands.reshape(N, D)
    b1 = enc_b1.reshape(1, H)
    b2 = enc_b2.reshape(1, L)
    db1 = dec_b1.reshape(1, H)
    db2 = dec_b2.reshape(1, D)

    row_spec = lambda c: pl.BlockSpec((R, c), lambda i: (i, 0))
    full2 = lambda a, b: pl.BlockSpec((a, b), lambda i: (0, 0))

    out_shapes = (
        jax.ShapeDtypeStruct((N, D), jnp.float32),
        jax.ShapeDtypeStruct((N, L), jnp.float32),
        jax.ShapeDtypeStruct((N, L), jnp.float32),
        jax.ShapeDtypeStruct((N, G), jnp.int32),
        jax.ShapeDtypeStruct((1, 1), jnp.float32),
    )

    bands_hat, z_e, z_q, idx, loss_sum = pl.pallas_call(
        functools.partial(_fused_body, G=G, K=K, GD=GD),
        grid=grid,
        in_specs=[
            row_spec(D),
            full2(D, H), full2(1, H), full2(H, L), full2(1, L),
            pl.BlockSpec((G, K, GD), lambda i: (0, 0, 0)),
            full2(L, H), full2(1, H), full2(H, D), full2(1, D),
        ],
        out_specs=(
            row_spec(D), row_spec(L), row_spec(L),
            pl.BlockSpec((R, G), lambda i: (i, 0)),
            pl.BlockSpec((1, 1), lambda i: (0, 0)),
        ),
        out_shape=out_shapes,
        scratch_shapes=[
            pltpu.VMEM((G, K, GD), jnp.float32),
            pltpu.VMEM((G, K), jnp.float32),
            pltpu.VMEM((G, K, GD + 1), jnp.float32),
        ],
    )(x, enc_w1, b1, enc_w2, b2, codebooks, dec_w1, db1, dec_w2, db2)

    vq_loss = (2.0 * beta / (N * GD)) * loss_sum[0, 0]
    return (bands_hat.reshape(B, T, D), z_e.reshape(B, T, L),
            z_q.reshape(B, T, L), idx.reshape(B, T, G), vq_loss)


# speedup vs baseline: 8.7844x; 1.0186x over previous
import functools

import jax
import jax.numpy as jnp
from jax.experimental import pallas as pl
from jax.experimental.pallas import tpu as pltpu


def _fused_body(x_ref, w1_ref, b1_ref, w2_ref, b2_ref, cb_ref,
                dw1_ref, db1_ref, dw2_ref, db2_ref,
                bh_ref, ze_ref, zq_ref, idx_ref, loss_ref,
                cb2_ref, cbn_ref, aug_ref,
                *, G, K, GD):
    @pl.when(pl.program_id(0) == 0)
    def _prep():
        cb = cb_ref[...]
        cb2_ref[...] = cb * 2.0
        cbn_ref[...] = jnp.sum(cb * cb, axis=1)
        aug_ref[:, :GD, :] = cb
        aug_ref[:, GD:, :] = jax.lax.broadcasted_iota(
            jnp.int32, (G, 1, K), 2).astype(jnp.float32)

    x = x_ref[...]

    h = jnp.maximum(jnp.dot(x, w1_ref[...]) + b1_ref[...], 0.0)
    z_e = jnp.dot(h, w2_ref[...]) + b2_ref[...]
    ze_ref[...] = z_e

    zq_parts = []
    idx_parts = []
    for g in range(G):
        zg = z_e[:, g * GD:(g + 1) * GD]
        s2 = jax.lax.dot_general(zg, cb2_ref[g], (((1,), (0,)), ((), ())))
        a = s2 - cbn_ref[g][None, :]
        m = jnp.max(a, axis=1, keepdims=True)
        maskf = jnp.where(a == m, 1.0, 0.0)
        r = jax.lax.dot_general(maskf, aug_ref[g],
                                (((1,), (1,)), ((), ())))
        zq_parts.append(r[:, :GD])
        idx_parts.append(r[:, GD:])

    z_q = jnp.concatenate(zq_parts, axis=1)
    zq_ref[...] = z_q
    idx_ref[...] = jnp.concatenate(idx_parts, axis=1).astype(jnp.int32)
    dz = z_q - z_e
    block_sq = jnp.sum(dz * dz)

    @pl.when(pl.program_id(0) == 0)
    def _init():
        loss_ref[...] = jnp.zeros((1, 1), jnp.float32)

    loss_ref[...] += block_sq[None, None]

    hd = jnp.maximum(jnp.dot(z_q, dw1_ref[...]) + db1_ref[...], 0.0)
    bh_ref[...] = jnp.dot(hd, dw2_ref[...]) + db2_ref[...]


def kernel(bands, enc_w1, enc_b1, enc_w2, enc_b2, codebooks,
           dec_w1, dec_b1, dec_w2, dec_b2):
    B, T, D = bands.shape
    N = B * T
    G, K, GD = codebooks.shape
    H = enc_w1.shape[1]
    L = enc_w2.shape[1]
    beta = 0.25

    R = min(2048, N)
    assert N % R == 0
    grid = (N // R,)

    x = bands.reshape(N, D)
    b1 = enc_b1.reshape(1, H)
    b2 = enc_b2.reshape(1, L)
    db1 = dec_b1.reshape(1, H)
    db2 = dec_b2.reshape(1, D)

    row_spec = lambda c: pl.BlockSpec((R, c), lambda i: (i, 0))
    full2 = lambda a, b: pl.BlockSpec((a, b), lambda i: (0, 0))

    out_shapes = (
        jax.ShapeDtypeStruct((N, D), jnp.float32),
        jax.ShapeDtypeStruct((N, L), jnp.float32),
        jax.ShapeDtypeStruct((N, L), jnp.float32),
        jax.ShapeDtypeStruct((N, G), jnp.int32),
        jax.ShapeDtypeStruct((1, 1), jnp.float32),
    )

    bands_hat, z_e, z_q, idx, loss_sum = pl.pallas_call(
        functools.partial(_fused_body, G=G, K=K, GD=GD),
        grid=grid,
        in_specs=[
            row_spec(D),
            full2(D, H), full2(1, H), full2(H, L), full2(1, L),
            pl.BlockSpec((G, GD, K), lambda i: (0, 0, 0)),
            full2(L, H), full2(1, H), full2(H, D), full2(1, D),
        ],
        out_specs=(
            row_spec(D), row_spec(L), row_spec(L),
            pl.BlockSpec((R, G), lambda i: (i, 0)),
            pl.BlockSpec((1, 1), lambda i: (0, 0)),
        ),
        out_shape=out_shapes,
        scratch_shapes=[
            pltpu.VMEM((G, GD, K), jnp.float32),
            pltpu.VMEM((G, K), jnp.float32),
            pltpu.VMEM((G, GD + 1, K), jnp.float32),
        ],
    )(x, enc_w1, b1, enc_w2, b2, codebooks.transpose(0, 2, 1),
      dec_w1, db1, dec_w2, db2)

    vq_loss = (2.0 * beta / (N * GD)) * loss_sum[0, 0]
    return (bands_hat.reshape(B, T, D), z_e.reshape(B, T, L),
            z_q.reshape(B, T, L), idx.reshape(B, T, G), vq_loss)
